# baseline (speedup 1.0000x reference)
import functools

import jax
import jax.numpy as jnp
from jax import lax
from jax.experimental import pallas as pl
from jax.experimental.pallas import tpu as pltpu
from jax.experimental.pallas import tpu_sc as plsc

N_NODES = 10000
N_EDGES = 320000
D = 128

NC = 2
NS = 16
L = 16
NW = NC * NS
E_PER_W = 10240
E_PAD = E_PER_W * NW - N_EDGES
SUB = 64
SEG = 512
N_SEG = E_PER_W // SEG
N_SUB = SEG // SUB
NBUF = 4
NSTG = 3
ROWS_PER_TILE = N_NODES // NS
DUMP_ROWS = 632
DUMP_LAST = N_NODES - (NS - 1) * DUMP_ROWS


def _bcast_lane(vec, j):
    idx = jnp.full((L, 1), j, dtype=jnp.int32)
    return lax.gather(
        vec, idx,
        dimension_numbers=lax.GatherDimensionNumbers(
            offset_dims=(), collapsed_slice_dims=(0,), start_index_map=(0,)),
        slice_sizes=(1,),
        mode=lax.GatherScatterMode.PROMISE_IN_BOUNDS)


def _sc_aggregate(ego, src, dst3, w):
    mesh = plsc.VectorSubcoreMesh(core_axis_name="c", subcore_axis_name="s")

    @functools.partial(
        pl.kernel,
        out_type=jax.ShapeDtypeStruct((NC, N_NODES, D), jnp.float32),
        mesh=mesh,
        scratch_types=[
            pltpu.VMEM((NSTG * N_SUB, SUB), jnp.int32),
            pltpu.VMEM((NSTG * N_SUB, SUB), jnp.int32),
            pltpu.VMEM((NSTG * N_SUB, SUB), jnp.float32),
            pltpu.VMEM((NBUF, SUB, D), jnp.float32),
            pltpu.VMEM_SHARED((N_NODES, D), jnp.float32),
            pltpu.SemaphoreType.DMA,
            pltpu.SemaphoreType.DMA,
            pltpu.SemaphoreType.DMA,
            pltpu.SemaphoreType.DMA,
            pltpu.SemaphoreType.DMA,
            pltpu.SemaphoreType.DMA,
            pltpu.SemaphoreType.DMA,
            pltpu.SemaphoreType.DMA,
            pltpu.SemaphoreType.DMA,
            pltpu.SemaphoreType.DMA,
            pltpu.SemaphoreType.DMA,
        ],
    )
    def k(ego_hbm, src_hbm, dst_hbm, w_hbm, out_hbm,
          src_v, dst_v, w_v, rows_v, acc_sh,
          sg0, sg1, sg2, sg3, ss0, ss1, ss2, ss3, st0, st1, st2):
        sem_g = [sg0, sg1, sg2, sg3]
        sem_s = [ss0, ss1, ss2, ss3]
        sem_t = [st0, st1, st2]
        cid = lax.axis_index("c")
        sid = lax.axis_index("s")
        wid = cid * NS + sid

        def start_gather(sb, c, rb):
            pltpu.async_copy(
                ego_hbm.at[src_v.at[sb * N_SUB + c]],
                rows_v.at[rb], sem_g[rb])

        def wait_gather(rb):
            pltpu.make_async_copy(
                ego_hbm.at[src_v.at[0]],
                rows_v.at[rb], sem_g[rb]).wait()

        def start_scatter(sb, c, rb):
            pltpu.async_copy(rows_v.at[rb],
                             acc_sh.at[dst_v.at[sb * N_SUB + c]],
                             sem_s[rb], add=True)

        def wait_scatter(rb):
            pltpu.make_async_copy(rows_v.at[rb], acc_sh.at[dst_v.at[0]],
                                  sem_s[rb]).wait()

        def start_stage(s1, tb):
            sl = pl.ds(tb * N_SUB, N_SUB)
            pltpu.async_copy(src_hbm.at[wid, s1], src_v.at[sl], sem_t[tb])
            pltpu.async_copy(dst_hbm.at[wid, s1], dst_v.at[sl], sem_t[tb])
            pltpu.async_copy(w_hbm.at[wid, s1], w_v.at[sl], sem_t[tb])

        def wait_stage(tb):
            sl = pl.ds(tb * N_SUB, N_SUB)
            pltpu.make_async_copy(src_hbm.at[wid, 0], src_v.at[sl],
                                  sem_t[tb]).wait()
            pltpu.make_async_copy(dst_hbm.at[wid, 0], dst_v.at[sl],
                                  sem_t[tb]).wait()
            pltpu.make_async_copy(w_hbm.at[wid, 0], w_v.at[sl],
                                  sem_t[tb]).wait()

        def scale(sb, c, rb):
            def grp_body(g, _):
                wv = w_v[sb * N_SUB + c, pl.ds(g * L, L)]

                @plsc.parallel_loop(0, L, 1, unroll=4)
                def _edge(j):
                    wj = _bcast_lane(wv, j)
                    e = g * L + j
                    for fb in range(D // L):
                        x = rows_v[rb, e, pl.ds(fb * L, L)]
                        rows_v[rb, e, pl.ds(fb * L, L)] = x * wj
                return 0
            lax.fori_loop(0, SUB // L, grp_body, 0)

        pltpu.sync_copy(src_hbm.at[wid, 0], src_v.at[pl.ds(0, N_SUB)])
        pltpu.sync_copy(dst_hbm.at[wid, 0], dst_v.at[pl.ds(0, N_SUB)])
        pltpu.sync_copy(w_hbm.at[wid, 0], w_v.at[pl.ds(0, N_SUB)])
        start_gather(0, 0, 0)
        start_gather(0, 1, 1)
        start_stage(1, 1)

        def zero_body(i, _):
            r = i // (D // L)
            c = (i % (D // L)) * L
            rows_v[3, r, pl.ds(c, L)] = jnp.zeros((L,), jnp.float32)
            return 0
        lax.fori_loop(0, SUB * (D // L), zero_body, 0)
        r_base = sid * ROWS_PER_TILE
        for t in range(ROWS_PER_TILE // SUB):
            pltpu.sync_copy(rows_v.at[3],
                            acc_sh.at[pl.ds(r_base + t * SUB, SUB)])
        pltpu.sync_copy(
            rows_v.at[3, pl.ds(0, ROWS_PER_TILE % SUB)],
            acc_sh.at[pl.ds(r_base + (ROWS_PER_TILE // SUB) * SUB,
                            ROWS_PER_TILE % SUB)])
        plsc.subcore_barrier()
        pltpu.async_copy(rows_v.at[3], acc_sh.at[dst_v.at[0]],
                         sem_s[2], add=True)
        pltpu.async_copy(rows_v.at[3], acc_sh.at[dst_v.at[0]],
                         sem_s[3], add=True)

        def seg_body(s, _):
            sb = s % NSTG
            sbn = (s + 1) % NSTG

            def chunk(c, rb, cross):
                wait_gather(rb)
                wait_scatter((rb + 2) % 4)
                if cross is None:
                    start_gather(sb, c + 2, (rb + 2) % 4)
                elif cross >= 0:
                    @pl.when(s < N_SEG - 1)
                    def _x():
                        start_gather(sbn, jnp.int32(cross), (rb + 2) % 4)
                scale(sb, c, rb)
                start_scatter(sb, c, rb)

            def quad(jj, _):
                c0 = 4 * jj
                for i in range(4):
                    chunk(c0 + i, i, None)
                return 0
            lax.fori_loop(0, N_SUB // 4 - 1, quad, 0)

            @pl.when(s < N_SEG - 2)
            def _stage_next():
                for tb in range(NSTG):
                    @pl.when((s + 2) % NSTG == tb)
                    def _st():
                        start_stage(s + 2, tb)

            chunk(jnp.int32(N_SUB - 4), 0, None)
            chunk(jnp.int32(N_SUB - 3), 1, None)

            @pl.when(s < N_SEG - 1)
            def _wait_next_stage():
                for tb in range(NSTG):
                    @pl.when(sbn == tb)
                    def _wt():
                        wait_stage(tb)

            chunk(jnp.int32(N_SUB - 2), 2, 0)
            chunk(jnp.int32(N_SUB - 1), 3, 1)
            return 0
        lax.fori_loop(0, N_SEG, seg_body, 0)

        wait_scatter(2)
        wait_scatter(3)
        plsc.subcore_barrier()

        @pl.when(sid < NS - 1)
        def _dump_main():
            r0 = sid * DUMP_ROWS
            pltpu.sync_copy(acc_sh.at[pl.ds(r0, DUMP_ROWS)],
                            out_hbm.at[cid, pl.ds(r0, DUMP_ROWS)])

        @pl.when(sid == NS - 1)
        def _dump_last():
            r0 = (NS - 1) * DUMP_ROWS
            pltpu.sync_copy(acc_sh.at[pl.ds(r0, DUMP_LAST)],
                            out_hbm.at[cid, pl.ds(r0, DUMP_LAST)])

    return k(ego, src, dst3, w)


def _norm_kernel(p_ref, o_ref):
    h = p_ref[0] + p_ref[1]
    n2 = jnp.sum(h * h, axis=1, keepdims=True)
    n = jnp.sqrt(n2)
    o_ref[...] = h / jnp.maximum(n, 1e-12)


def _combine_normalize(partials):
    blk = N_NODES // 10
    return pl.pallas_call(
        _norm_kernel,
        grid=(N_NODES // blk,),
        in_specs=[pl.BlockSpec((NC, blk, D), lambda i: (0, i, 0))],
        out_specs=pl.BlockSpec((blk, D), lambda i: (i, 0)),
        out_shape=jax.ShapeDtypeStruct((N_NODES, D), jnp.float32),
    )(partials)


@jax.jit
def kernel(ego_embedding, edge_index, edge_weight):
    pad_i = jnp.arange(E_PAD, dtype=jnp.int32) % N_NODES
    src = jnp.concatenate(
        [edge_index[0].astype(jnp.int32), pad_i]).reshape(
            NW, N_SEG, N_SUB, SUB)
    dst = jnp.concatenate(
        [edge_index[1].astype(jnp.int32), pad_i]).reshape(
            NW, N_SEG, N_SUB, SUB)
    w = jnp.concatenate(
        [edge_weight, jnp.zeros((E_PAD,), jnp.float32)]).reshape(
            NW, N_SEG, N_SUB, SUB)
    partials = _sc_aggregate(ego_embedding, src, dst, w)
    return _combine_normalize(partials)


# speedup vs baseline: 1.0170x; 1.0170x over previous
import functools

import jax
import jax.numpy as jnp
from jax import lax
from jax.experimental import pallas as pl
from jax.experimental.pallas import tpu as pltpu
from jax.experimental.pallas import tpu_sc as plsc

N_NODES = 10000
N_EDGES = 320000
D = 128

NC = 2
NS = 16
L = 16
NW = NC * NS
E_PER_W = 10240
E_PAD = E_PER_W * NW - N_EDGES
SUB = 64
SEG = 512
N_SEG = E_PER_W // SEG
N_SUB = SEG // SUB
NBUF = 4
NSTG = 3
ROWS_PER_TILE = N_NODES // NS
DUMP_ROWS = 632
DUMP_LAST = N_NODES - (NS - 1) * DUMP_ROWS


def _bcast_lane(vec, j):
    idx = jnp.full((L, 1), j, dtype=jnp.int32)
    return lax.gather(
        vec, idx,
        dimension_numbers=lax.GatherDimensionNumbers(
            offset_dims=(), collapsed_slice_dims=(0,), start_index_map=(0,)),
        slice_sizes=(1,),
        mode=lax.GatherScatterMode.PROMISE_IN_BOUNDS)


def _sc_aggregate(ego, src, dst3, w):
    mesh = plsc.VectorSubcoreMesh(core_axis_name="c", subcore_axis_name="s")

    @functools.partial(
        pl.kernel,
        out_type=jax.ShapeDtypeStruct((NC, N_NODES, D), jnp.float32),
        mesh=mesh,
        scratch_types=[
            pltpu.VMEM((NSTG * N_SUB, SUB), jnp.int32),
            pltpu.VMEM((NSTG * N_SUB, SUB), jnp.int32),
            pltpu.VMEM((NSTG * N_SUB, SUB), jnp.float32),
            pltpu.VMEM((NBUF, SUB, D), jnp.float32),
            pltpu.VMEM_SHARED((N_NODES, D), jnp.float32),
            pltpu.SemaphoreType.DMA,
            pltpu.SemaphoreType.DMA,
            pltpu.SemaphoreType.DMA,
            pltpu.SemaphoreType.DMA,
            pltpu.SemaphoreType.DMA,
            pltpu.SemaphoreType.DMA,
            pltpu.SemaphoreType.DMA,
            pltpu.SemaphoreType.DMA,
            pltpu.SemaphoreType.DMA,
            pltpu.SemaphoreType.DMA,
            pltpu.SemaphoreType.DMA,
        ],
    )
    def k(ego_hbm, src_hbm, dst_hbm, w_hbm, out_hbm,
          src_v, dst_v, w_v, rows_v, acc_sh,
          sg0, sg1, sg2, sg3, ss0, ss1, ss2, ss3, st0, st1, st2):
        sem_g = [sg0, sg1, sg2, sg3]
        sem_s = [ss0, ss1, ss2, ss3]
        sem_t = [st0, st1, st2]
        cid = lax.axis_index("c")
        sid = lax.axis_index("s")
        wid = cid * NS + sid

        def start_gather(sb, c, rb):
            pltpu.async_copy(
                ego_hbm.at[src_v.at[sb * N_SUB + c]],
                rows_v.at[rb], sem_g[rb])

        def wait_gather(rb):
            pltpu.make_async_copy(
                ego_hbm.at[src_v.at[0]],
                rows_v.at[rb], sem_g[rb]).wait()

        def start_scatter(sb, c, rb):
            pltpu.async_copy(rows_v.at[rb],
                             acc_sh.at[dst_v.at[sb * N_SUB + c]],
                             sem_s[rb], add=True)

        def wait_scatter(rb):
            pltpu.make_async_copy(rows_v.at[rb], acc_sh.at[dst_v.at[0]],
                                  sem_s[rb]).wait()

        def start_stage(s1, tb):
            sl = pl.ds(tb * N_SUB, N_SUB)
            pltpu.async_copy(src_hbm.at[wid, s1], src_v.at[sl], sem_t[tb])
            pltpu.async_copy(dst_hbm.at[wid, s1], dst_v.at[sl], sem_t[tb])
            pltpu.async_copy(w_hbm.at[wid, s1], w_v.at[sl], sem_t[tb])

        def wait_stage(tb):
            sl = pl.ds(tb * N_SUB, N_SUB)
            pltpu.make_async_copy(src_hbm.at[wid, 0], src_v.at[sl],
                                  sem_t[tb]).wait()
            pltpu.make_async_copy(dst_hbm.at[wid, 0], dst_v.at[sl],
                                  sem_t[tb]).wait()
            pltpu.make_async_copy(w_hbm.at[wid, 0], w_v.at[sl],
                                  sem_t[tb]).wait()

        def scale(sb, c, rb):
            def grp_body(g, _):
                wv = w_v[sb * N_SUB + c, pl.ds(g * L, L)]

                @plsc.parallel_loop(0, L, 1, unroll=4)
                def _edge(j):
                    wj = _bcast_lane(wv, j)
                    e = g * L + j
                    for fb in range(D // L):
                        x = rows_v[rb, e, pl.ds(fb * L, L)]
                        rows_v[rb, e, pl.ds(fb * L, L)] = x * wj
                return 0
            lax.fori_loop(0, SUB // L, grp_body, 0)

        pltpu.sync_copy(src_hbm.at[wid, 0], src_v.at[pl.ds(0, N_SUB)])
        pltpu.sync_copy(dst_hbm.at[wid, 0], dst_v.at[pl.ds(0, N_SUB)])
        pltpu.sync_copy(w_hbm.at[wid, 0], w_v.at[pl.ds(0, N_SUB)])
        start_gather(0, 0, 0)
        start_gather(0, 1, 1)
        start_stage(1, 1)

        def zero_body(i, _):
            r = i // (D // L)
            c = (i % (D // L)) * L
            rows_v[3, r, pl.ds(c, L)] = jnp.zeros((L,), jnp.float32)
            return 0
        lax.fori_loop(0, SUB * (D // L), zero_body, 0)
        r_base = sid * ROWS_PER_TILE
        for t in range(ROWS_PER_TILE // SUB):
            pltpu.sync_copy(rows_v.at[3],
                            acc_sh.at[pl.ds(r_base + t * SUB, SUB)])
        pltpu.sync_copy(
            rows_v.at[3, pl.ds(0, ROWS_PER_TILE % SUB)],
            acc_sh.at[pl.ds(r_base + (ROWS_PER_TILE // SUB) * SUB,
                            ROWS_PER_TILE % SUB)])
        plsc.subcore_barrier()
        pltpu.async_copy(rows_v.at[3], acc_sh.at[dst_v.at[0]],
                         sem_s[2], add=True)
        pltpu.async_copy(rows_v.at[3], acc_sh.at[dst_v.at[0]],
                         sem_s[3], add=True)

        def seg_body(s, _):
            sb = s % NSTG
            sbn = (s + 1) % NSTG

            def chunk(c, rb, cross):
                wait_gather(rb)
                wait_scatter((rb + 2) % 4)
                if cross is None:
                    start_gather(sb, c + 2, (rb + 2) % 4)
                elif cross >= 0:
                    @pl.when(s < N_SEG - 1)
                    def _x():
                        start_gather(sbn, jnp.int32(cross), (rb + 2) % 4)
                scale(sb, c, rb)
                start_scatter(sb, c, rb)

            def quad(jj, _):
                c0 = 4 * jj
                for i in range(4):
                    chunk(c0 + i, i, None)
                return 0
            lax.fori_loop(0, N_SUB // 4 - 1, quad, 0)

            @pl.when(s < N_SEG - 2)
            def _stage_next():
                for tb in range(NSTG):
                    @pl.when((s + 2) % NSTG == tb)
                    def _st():
                        start_stage(s + 2, tb)

            chunk(jnp.int32(N_SUB - 4), 0, None)
            chunk(jnp.int32(N_SUB - 3), 1, None)

            @pl.when(s < N_SEG - 1)
            def _wait_next_stage():
                for tb in range(NSTG):
                    @pl.when(sbn == tb)
                    def _wt():
                        wait_stage(tb)

            chunk(jnp.int32(N_SUB - 2), 2, 0)
            chunk(jnp.int32(N_SUB - 1), 3, 1)
            return 0
        lax.fori_loop(0, N_SEG, seg_body, 0)

        wait_scatter(2)
        wait_scatter(3)
        plsc.subcore_barrier()

        @pl.when(sid < NS - 1)
        def _dump_main():
            r0 = sid * DUMP_ROWS
            pltpu.sync_copy(acc_sh.at[pl.ds(r0, DUMP_ROWS)],
                            out_hbm.at[cid, pl.ds(r0, DUMP_ROWS)])

        @pl.when(sid == NS - 1)
        def _dump_last():
            r0 = (NS - 1) * DUMP_ROWS
            pltpu.sync_copy(acc_sh.at[pl.ds(r0, DUMP_LAST)],
                            out_hbm.at[cid, pl.ds(r0, DUMP_LAST)])

    return k(ego, src, dst3, w)


def _norm_kernel(p_ref, o_ref):
    h = p_ref[0] + p_ref[1]
    n2 = jnp.sum(h * h, axis=1, keepdims=True)
    n = jnp.sqrt(n2)
    o_ref[...] = h / jnp.maximum(n, 1e-12)


def _combine_normalize(partials):
    return pl.pallas_call(
        _norm_kernel,
        out_shape=jax.ShapeDtypeStruct((N_NODES, D), jnp.float32),
    )(partials)


@jax.jit
def kernel(ego_embedding, edge_index, edge_weight):
    pad_i = jnp.arange(E_PAD, dtype=jnp.int32) % N_NODES
    src = jnp.concatenate(
        [edge_index[0].astype(jnp.int32), pad_i]).reshape(
            NW, N_SEG, N_SUB, SUB)
    dst = jnp.concatenate(
        [edge_index[1].astype(jnp.int32), pad_i]).reshape(
            NW, N_SEG, N_SUB, SUB)
    w = jnp.concatenate(
        [edge_weight, jnp.zeros((E_PAD,), jnp.float32)]).reshape(
            NW, N_SEG, N_SUB, SUB)
    partials = _sc_aggregate(ego_embedding, src, dst, w)
    return _combine_normalize(partials)
